# baseline (speedup 1.0000x reference)
import jax
import jax.numpy as jnp
from jax import lax
from jax.experimental import pallas as pl
from jax.experimental.pallas import tpu as pltpu
from jax.experimental.pallas import tpu_sc as plsc

NC = 2
NS = 16
L = 16
NW = NC * NS
K = 32
G = 128 // K
CH = 128
HALF = 256


def kernel(data_u, data_i, u_table, i_table):
    B = data_u.shape[0]
    bw = B // NW
    mesh = plsc.VectorSubcoreMesh(core_axis_name="c", subcore_axis_name="s")

    @pl.kernel(
        mesh=mesh,
        out_type=jax.ShapeDtypeStruct((B,), jnp.float32),
        scratch_types=[
            pltpu.VMEM((bw + L,), jnp.int32),
            pltpu.VMEM((bw + L,), jnp.int32),
            pltpu.VMEM((bw,), jnp.int32),
            pltpu.VMEM((bw,), jnp.int32),
            pltpu.VMEM((HALF, 128), jnp.float32),
            pltpu.VMEM((HALF, 128), jnp.float32),
            pltpu.VMEM((bw + L,), jnp.float32),
            pltpu.SemaphoreType.DMA,
            pltpu.SemaphoreType.DMA,
        ],
        compiler_params=pltpu.CompilerParams(needs_layout_passes=False),
    )
    def mf(du, di, ut, it, out, idx_u, idx_i, gid_u, gid_i, u_g, i_g, out_v,
           sem_u, sem_i):
        wid = lax.axis_index("s") * NC + lax.axis_index("c")
        base = wid * bw

        pltpu.sync_copy(du.at[pl.ds(base, bw)], idx_u.at[pl.ds(0, bw)])
        pltpu.sync_copy(di.at[pl.ds(base, bw)], idx_i.at[pl.ds(0, bw)])

        @plsc.parallel_loop(0, bw // L, 1, unroll=4)
        def _(j):
            gid_u[pl.ds(j * L, L)] = idx_u[pl.ds(j * L, L)] >> 2
            gid_i[pl.ds(j * L, L)] = idx_i[pl.ds(j * L, L)] >> 2

        last_lane = lax.iota(jnp.int32, L) == (L - 1)

        for h in range(bw // HALF):
            copies = []
            for c in range(HALF // CH):
                s = h * HALF + c * CH
                copies.append(pltpu.async_copy(
                    ut.at[gid_u.at[pl.ds(s, CH)]],
                    u_g.at[pl.ds(c * CH, CH)], sem_u))
                copies.append(pltpu.async_copy(
                    it.at[gid_i.at[pl.ds(s, CH)]],
                    i_g.at[pl.ds(c * CH, CH)], sem_i))
            for cp in copies:
                cp.wait()

            @plsc.parallel_loop(0, HALF, 1, unroll=8)
            def _(r):
                ra = h * HALF + r
                su = (idx_u[pl.ds(ra, L)][0] & (G - 1)) * K
                si = (idx_i[pl.ds(ra, L)][0] & (G - 1)) * K
                p = (u_g[r, pl.ds(su, L)] * i_g[r, pl.ds(si, L)] +
                     u_g[r, pl.ds(su + L, L)] * i_g[r, pl.ds(si + L, L)])
                sacc = plsc.cumsum(p)
                plsc.store_compressed(out_v.at[pl.ds(ra, L)], sacc,
                                      mask=last_lane)

        pltpu.sync_copy(out_v.at[pl.ds(0, bw)], out.at[pl.ds(base, bw)])

    ut4 = u_table.reshape(u_table.shape[0] // G, 128)
    it4 = i_table.reshape(i_table.shape[0] // G, 128)
    return mf(data_u.astype(jnp.int32), data_i.astype(jnp.int32), ut4, it4)
